# Initial kernel scaffold; baseline (speedup 1.0000x reference)
import jax
import jax.numpy as jnp
from jax.experimental import pallas as pl


def kernel(patches, memory):
    raise NotImplementedError("write your pallas kernel here")


# speedup vs baseline: 4.4653x; 4.4653x over previous
import math

import numpy as np
import jax
import jax.numpy as jnp
from jax import lax
from jax.experimental import pallas as pl
from jax.experimental.pallas import tpu as pltpu

FMAP = 28
IMG = 224
_BIG = 2**30


def _pick_block(n, cands):
    for c in cands:
        if n % c == 0 and c % 8 == 0:
            return c
    return n


def _resize_mat():
    r = np.zeros((IMG, FMAP), np.float32)
    for i in range(IMG):
        x = (i + 0.5) * FMAP / IMG - 0.5
        j0 = int(np.floor(x))
        t = x - j0
        r[i, min(max(j0, 0), FMAP - 1)] += 1.0 - t
        r[i, min(max(j0 + 1, 0), FMAP - 1)] += t
    return jnp.asarray(r)


def _pass1(patches, memory, b1):
    q, d = patches.shape
    n = memory.shape[0]
    g1 = n // b1

    def body(p_ref, m_ref, dist_ref, sidx_ref, jstar_ref, minv_ref, argv_ref):
        b = pl.program_id(0)

        @pl.when(b == 0)
        def _init():
            minv_ref[...] = jnp.full((q, 1), jnp.inf, jnp.float32)
            argv_ref[...] = jnp.zeros((q, 1), jnp.int32)

        p = p_ref[...]
        m = m_ref[...]
        ones = jnp.ones((1, d), jnp.float32)
        mn = lax.dot_general(ones, m * m, (((1,), (1,)), ((), ())),
                             preferred_element_type=jnp.float32)
        dot = lax.dot_general(p, m, (((1,), (1,)), ((), ())),
                              preferred_element_type=jnp.float32)
        s = mn - 2.0 * dot
        bmin = jnp.min(s, axis=1, keepdims=True)
        colid = lax.broadcasted_iota(jnp.int32, (q, b1), 1) + b * b1
        barg = jnp.min(jnp.where(s == bmin, colid, _BIG), axis=1,
                       keepdims=True)
        prev = minv_ref[...]
        upd = bmin < prev
        minv_ref[...] = jnp.where(upd, bmin, prev)
        argv_ref[...] = jnp.where(upd, barg, argv_ref[...])

        @pl.when(b == g1 - 1)
        def _fin():
            pn = jnp.sum(p * p, axis=1, keepdims=True)
            d2 = minv_ref[...] + pn
            dist = jnp.sqrt(jnp.maximum(d2, 1e-12))
            dist_ref[...] = dist
            mx = jnp.max(dist)
            rowid = lax.broadcasted_iota(jnp.int32, (q, 1), 0)
            sidx = jnp.min(jnp.where(dist == mx, rowid, _BIG))
            sidx_ref[0, 0] = sidx
            jstar_ref[0, 0] = jnp.sum(
                jnp.where(rowid == sidx, argv_ref[...], 0))

    return pl.pallas_call(
        body,
        grid=(g1,),
        in_specs=[
            pl.BlockSpec((q, d), lambda b: (0, 0)),
            pl.BlockSpec((b1, d), lambda b: (b, 0)),
        ],
        out_specs=[
            pl.BlockSpec((q, 1), lambda b: (0, 0)),
            pl.BlockSpec(memory_space=pltpu.SMEM),
            pl.BlockSpec(memory_space=pltpu.SMEM),
        ],
        out_shape=[
            jax.ShapeDtypeStruct((q, 1), jnp.float32),
            jax.ShapeDtypeStruct((1, 1), jnp.int32),
            jax.ShapeDtypeStruct((1, 1), jnp.int32),
        ],
        scratch_shapes=[
            pltpu.VMEM((q, 1), jnp.float32),
            pltpu.VMEM((q, 1), jnp.int32),
        ],
    )(patches, memory)


def _pass2(m_star, memory, b2):
    n, d = memory.shape
    g2 = n // b2

    def body(ms_ref, m_ref, i1_ref, i2_ref, rbuf_ref):
        b = pl.program_id(0)
        m = m_ref[...]
        ms = ms_ref[...]
        ones = jnp.ones((1, d), jnp.float32)
        mn = lax.dot_general(ones, m * m, (((1,), (1,)), ((), ())),
                             preferred_element_type=jnp.float32)
        dot = lax.dot_general(ms, m, (((1,), (1,)), ((), ())),
                              preferred_element_type=jnp.float32)
        rbuf_ref[pl.ds(b, 1), :] = mn - 2.0 * dot

        @pl.when(b == g2 - 1)
        def _fin():
            v = rbuf_ref[...]
            flat = (lax.broadcasted_iota(jnp.int32, (g2, b2), 0) * b2
                    + lax.broadcasted_iota(jnp.int32, (g2, b2), 1))
            idxs = []
            for _ in range(3):
                lo = jnp.min(v)
                ix = jnp.min(jnp.where(v == lo, flat, _BIG))
                idxs.append(ix)
                v = jnp.where(flat == ix, jnp.inf, v)
            i1_ref[0, 0] = idxs[1]
            i2_ref[0, 0] = idxs[2]

    return pl.pallas_call(
        body,
        grid=(g2,),
        in_specs=[
            pl.BlockSpec((1, d), lambda b: (0, 0)),
            pl.BlockSpec((b2, d), lambda b: (b, 0)),
        ],
        out_specs=[
            pl.BlockSpec(memory_space=pltpu.SMEM),
            pl.BlockSpec(memory_space=pltpu.SMEM),
        ],
        out_shape=[
            jax.ShapeDtypeStruct((1, 1), jnp.int32),
            jax.ShapeDtypeStruct((1, 1), jnp.int32),
        ],
        scratch_shapes=[pltpu.VMEM((g2, b2), jnp.float32)],
    )(m_star, memory)


def _epilogue(dist28, patches, n1, n2, sidx, rmat):
    q, d = patches.shape

    def body(dist_ref, p_ref, n1_ref, n2_ref, sidx_ref, r_ref,
             score_ref, segm_ref):
        dist = dist_ref[...]
        sstar = jnp.max(dist)
        p = p_ref[...]
        sidx = sidx_ref[0, 0]
        rowid = lax.broadcasted_iota(jnp.int32, (q, 1), 0)
        mte = jnp.sum(jnp.where(rowid == sidx, p, 0.0), axis=0,
                      keepdims=True)
        d1 = mte - n1_ref[...]
        d2_ = mte - n2_ref[...]
        wd1 = jnp.sqrt(jnp.sum(d1 * d1))
        wd2 = jnp.sqrt(jnp.sum(d2_ * d2_))
        nrm = jnp.float32(math.sqrt(d))
        w = 1.0 - jnp.exp(sstar / nrm) / (jnp.exp(wd1 / nrm)
                                          + jnp.exp(wd2 / nrm))
        score_ref[0, 0] = w * sstar
        r = r_ref[...]
        t1 = lax.dot_general(r, dist, (((1,), (0,)), ((), ())),
                             preferred_element_type=jnp.float32)
        segm_ref[...] = lax.dot_general(t1, r, (((1,), (1,)), ((), ())),
                                        preferred_element_type=jnp.float32)

    return pl.pallas_call(
        body,
        in_specs=[
            pl.BlockSpec(memory_space=pltpu.VMEM),
            pl.BlockSpec(memory_space=pltpu.VMEM),
            pl.BlockSpec(memory_space=pltpu.VMEM),
            pl.BlockSpec(memory_space=pltpu.VMEM),
            pl.BlockSpec(memory_space=pltpu.SMEM),
            pl.BlockSpec(memory_space=pltpu.VMEM),
        ],
        out_specs=[
            pl.BlockSpec(memory_space=pltpu.SMEM),
            pl.BlockSpec(memory_space=pltpu.VMEM),
        ],
        out_shape=[
            jax.ShapeDtypeStruct((1, 1), jnp.float32),
            jax.ShapeDtypeStruct((IMG, IMG), jnp.float32),
        ],
    )(dist28, patches, n1, n2, sidx, rmat)


def kernel(patches, memory):
    n, d = memory.shape
    b1 = _pick_block(n, (2000, 1000, 500, 400, 250, 200, 100, 80, 40, 16, 8))
    b2 = _pick_block(n, (4000, 2000, 1000, 500, 400, 200, 100, 80, 40, 16, 8))
    dist, sidx, jstar = _pass1(patches, memory, b1)
    m_star = lax.dynamic_slice(memory, (jstar[0, 0], 0), (1, d))
    i1, i2 = _pass2(m_star, memory, b2)
    n1 = lax.dynamic_slice(memory, (i1[0, 0], 0), (1, d))
    n2 = lax.dynamic_slice(memory, (i2[0, 0], 0), (1, d))
    dist28 = dist.reshape(FMAP, FMAP)
    score, segm = _epilogue(dist28, patches, n1, n2, sidx, _resize_mat())
    return (score[0, 0], segm.reshape(1, 1, IMG, IMG))
